# Initial kernel scaffold; baseline (speedup 1.0000x reference)
import jax
import jax.numpy as jnp
from jax.experimental import pallas as pl


def kernel(feats, edge_index, y, q, W_enc):
    raise NotImplementedError("write your pallas kernel here")


# speedup vs baseline: 9.6510x; 9.6510x over previous
import functools

import jax
import jax.numpy as jnp
from jax import lax
from jax.experimental import pallas as pl
from jax.experimental.pallas import tpu as pltpu
from jax.experimental.pallas import tpu_sc as plsc

N = 4096
D = 256
E = 65536
K = 512
RB = 128
NBLK = N // RB

NW = 32
TILE_ELEMS = N * N // NW
CH = 32768
NCHUNK = TILE_ELEMS // CH
NB = 1024
CAND = 640

_sc_mesh = functools.partial(
    plsc.VectorSubcoreMesh, core_axis_name="c", subcore_axis_name="s"
)
_SC_PARAMS = pltpu.CompilerParams(needs_layout_passes=False)


def _wid():
    return lax.axis_index("s") * 2 + lax.axis_index("c")


def _flip_kernel(feats_ref, w_ref, a_ref, flip_ref, loss_ref, h_ref):
    i = pl.program_id(0)

    @pl.when(i == 0)
    def _():
        h_ref[...] = jnp.tanh(
            lax.dot_general(feats_ref[...], w_ref[...], (((1,), (0,)), ((), ())))
        )
        loss_ref[0, 0] = 0.0

    h_blk = h_ref[pl.ds(i * RB, RB), :]
    s = lax.dot_general(h_blk, h_ref[...], (((1,), (1,)), ((), ())))
    sig = jax.nn.sigmoid(s)
    rows = i * RB + lax.broadcasted_iota(jnp.int32, (RB, N), 0)
    cols = lax.broadcasted_iota(jnp.int32, (RB, N), 1)
    sig = jnp.where(rows == cols, 0.0, sig)
    flip = jnp.abs(sig - a_ref[...])
    flip_ref[...] = flip
    loss_ref[0, 0] += jnp.sum(flip * flip)


def _compute_flip(feats, w_enc, a):
    return pl.pallas_call(
        _flip_kernel,
        grid=(NBLK,),
        in_specs=[
            pl.BlockSpec((N, D), lambda i: (0, 0)),
            pl.BlockSpec((D, D), lambda i: (0, 0)),
            pl.BlockSpec((RB, N), lambda i: (i, 0)),
        ],
        out_specs=[
            pl.BlockSpec((RB, N), lambda i: (i, 0)),
            pl.BlockSpec(memory_space=pltpu.SMEM),
        ],
        out_shape=[
            jax.ShapeDtypeStruct((N, N), jnp.float32),
            jax.ShapeDtypeStruct((1, 1), jnp.float32),
        ],
        scratch_shapes=[pltpu.VMEM((N, D), jnp.float32)],
    )(feats, w_enc, a)


def _scatter_edges_body(e_ref, a_hbm, ebuf_r, ebuf_c, idx2d, ones_v, sem):
    wid = _wid()
    base = wid * (E // NW)
    pltpu.sync_copy(e_ref.at[pl.ds(base, E // NW)], ebuf_r)
    pltpu.sync_copy(e_ref.at[pl.ds(E + base, E // NW)], ebuf_c)
    for j in range(8):
        ones_v[pl.ds(j * 16, 16)] = jnp.full((16,), 1.0, jnp.float32)
    for j in range(E // NW // 16):
        r16 = ebuf_r[pl.ds(j * 16, 16)]
        c16 = ebuf_c[pl.ds(j * 16, 16)]
        idx2d[j // 8, pl.ds((j % 8) * 16, 16)] = r16 * N + c16
    copies = []
    for k2 in range(16):
        copies.append(pltpu.async_copy(ones_v, a_hbm.at[idx2d.at[k2]], sem))
    for cp in copies:
        cp.wait()


def _scatter_edges(edge_flat, a_ref):
    pl.kernel(
        _scatter_edges_body,
        out_type=(),
        mesh=_sc_mesh(),
        compiler_params=_SC_PARAMS,
        scratch_types=[
            pltpu.VMEM((E // NW,), jnp.int32),
            pltpu.VMEM((E // NW,), jnp.int32),
            pltpu.VMEM((16, 128), jnp.int32),
            pltpu.VMEM((128,), jnp.float32),
            pltpu.SemaphoreType.DMA,
        ],
    )(edge_flat, a_ref)


def _accum_hist(hist_hbm, tmp, total):
    pltpu.sync_copy(hist_hbm, tmp)

    def body(w, _):
        def inner(j, _):
            total[pl.ds(j * 16, 16)] = total[pl.ds(j * 16, 16)] + tmp[
                pl.ds(w * NB + j * 16, 16)
            ]
            return 0

        return lax.fori_loop(0, NB // 16, inner, 0)

    lax.fori_loop(0, NW, body, 0)


def _zero(ref, n):
    def body(j, _):
        ref[pl.ds(j * 16, 16)] = jnp.zeros((16,), ref.dtype)
        return 0

    lax.fori_loop(0, n // 16, body, 0)


def _select_bucket(total, r_scalar):
    rvec = jnp.full((16,), 0, jnp.int32) + r_scalar

    def body(jj, carry):
        acc, cnt = carry
        j = NB // 16 - 1 - jj
        v = total[pl.ds(j * 16, 16)]
        rc = lax.rev(plsc.cumsum(lax.rev(v, (0,))), (0,))
        suf = rc + acc
        cnt = cnt + jnp.sum((suf >= rvec).astype(jnp.int32))
        acc = acc + jnp.sum(v)
        return acc, cnt

    _, cnt_ge = lax.fori_loop(0, NB // 16, body, (jnp.int32(0), jnp.int32(0)))
    b = cnt_ge - 1
    bvec = jnp.full((16,), 0, jnp.int32) + b

    def body2(j, s):
        v = total[pl.ds(j * 16, 16)]
        gi = j * 16 + lax.iota(jnp.int32, 16)
        s = s + jnp.sum(jnp.where(gi > bvec, v, 0))
        return s

    s_gt = lax.fori_loop(0, NB // 16, body2, jnp.int32(0))
    return b, s_gt


def _hist_pass_body(sh, nprev, flip_ref, *rest):
    prev_hbm = rest[:nprev]
    histo_hbm = rest[nprev]
    chunk, hist, tmp, total = rest[nprev + 1 : nprev + 5]
    wid = _wid()
    p = jnp.int32(0)
    r = jnp.int32(K)
    for ph in range(nprev):
        _zero(total, NB)
        _accum_hist(prev_hbm[ph], tmp, total)
        b, s_gt = _select_bucket(total, r)
        p = p * NB + b
        r = r - s_gt
    pvec = jnp.full((16,), 0, jnp.int32) + p
    _zero(hist, 16 * NB)
    laneoff = lax.iota(jnp.int32, 16) * NB
    ones16 = jnp.full((16,), 1, jnp.int32)
    base = wid * TILE_ELEMS

    def chunk_body(ci, _):
        pltpu.sync_copy(flip_ref.at[pl.ds(base + ci * CH, CH)], chunk)

        def vbody(j, _):
            x = chunk[pl.ds(j * 16, 16)]
            bits = plsc.bitcast(x, jnp.int32)
            pred = lax.shift_right_logical(bits, sh + 10) == pvec
            bucket = lax.bitwise_and(lax.shift_right_logical(bits, sh), NB - 1)
            plsc.addupdate_scatter(hist, [bucket + laneoff], ones16, mask=pred)
            return 0

        lax.fori_loop(0, CH // 16, vbody, 0)
        return 0

    lax.fori_loop(0, NCHUNK, chunk_body, 0)

    def fold(j, _):
        s = hist[pl.ds(j * 16, 16)]
        for l in range(1, 16):
            s = s + hist[pl.ds(l * NB + j * 16, 16)]
        tmp[pl.ds(j * 16, 16)] = s
        return 0

    lax.fori_loop(0, NB // 16, fold, 0)
    pltpu.sync_copy(tmp.at[pl.ds(0, NB)], histo_hbm.at[pl.ds(wid * NB, NB)])


def _hist_pass(sh, flip_flat, prev_hists):
    nprev = len(prev_hists)
    body = functools.partial(_hist_pass_body, sh, nprev)
    return pl.kernel(
        body,
        out_type=jax.ShapeDtypeStruct((NW * NB,), jnp.int32),
        mesh=_sc_mesh(),
        compiler_params=_SC_PARAMS,
        scratch_types=[
            pltpu.VMEM((CH,), jnp.float32),
            pltpu.VMEM((16 * NB,), jnp.int32),
            pltpu.VMEM((NW * NB,), jnp.int32),
            pltpu.VMEM((NB,), jnp.int32),
        ],
    )(flip_flat, *prev_hists)


def _compact_body(flip_ref, h1, h2, h3, gval_o, gidx_o, tidx_o, cnt_o,
                  chunk, gval, gidx, tidx, tmp, total, cbuf):
    wid = _wid()
    p = jnp.int32(0)
    r = jnp.int32(K)
    s_total = jnp.int32(0)
    for hh in (h1, h2, h3):
        _zero(total, NB)
        _accum_hist(hh, tmp, total)
        b, s_gt = _select_bucket(total, r)
        p = p * NB + b
        r = r - s_gt
        s_total = s_total + s_gt
    vstar_bits = p
    g_total = s_total
    t_need = jnp.int32(K) - g_total
    vstar = plsc.bitcast(jnp.full((16,), 0, jnp.int32) + vstar_bits, jnp.float32)
    base = wid * TILE_ELEMS
    iota16 = lax.iota(jnp.int32, 16)

    def chunk_body(ci, carry):
        pltpu.sync_copy(flip_ref.at[pl.ds(base + ci * CH, CH)], chunk)

        def vbody(j, carry):
            goff, toff = carry
            x = chunk[pl.ds(j * 16, 16)]
            m_gt = x > vstar
            m_eq = x == vstar
            n_gt = jnp.sum(m_gt.astype(jnp.int32))
            n_eq = jnp.sum(m_eq.astype(jnp.int32))

            def slow(c):
                goff, toff = c
                gidxv = base + ci * CH + j * 16 + iota16
                dg = goff + plsc.cumsum(m_gt.astype(jnp.int32)) - 1
                mg = m_gt & (dg < CAND - 16)
                plsc.store_scatter(gval, [dg], x, mask=mg)
                plsc.store_scatter(gidx, [dg], gidxv, mask=mg)
                dt = toff + plsc.cumsum(m_eq.astype(jnp.int32)) - 1
                mt = m_eq & (dt < K)
                plsc.store_scatter(tidx, [dt], gidxv, mask=mt)
                return goff + n_gt, toff + n_eq

            return lax.cond(n_gt + n_eq > 0, slow, lambda c: c, (goff, toff))

        return lax.fori_loop(0, CH // 16, vbody, carry)

    goff, toff = lax.fori_loop(0, NCHUNK, chunk_body, (jnp.int32(0), jnp.int32(0)))
    cvals = jnp.where(
        iota16 == 0, goff,
        jnp.where(iota16 == 1, jnp.minimum(toff, K),
                  jnp.where(iota16 == 2, g_total,
                            jnp.where(iota16 == 3, t_need, vstar_bits))))
    cbuf[pl.ds(0, 16)] = cvals
    pltpu.sync_copy(gval, gval_o.at[pl.ds(wid * CAND, CAND)])
    pltpu.sync_copy(gidx, gidx_o.at[pl.ds(wid * CAND, CAND)])
    pltpu.sync_copy(tidx, tidx_o.at[pl.ds(wid * CAND, CAND)])
    pltpu.sync_copy(cbuf, cnt_o.at[pl.ds(wid * 16, 16)])


def _compact(flip_flat, h1, h2, h3):
    return pl.kernel(
        _compact_body,
        out_type=(
            jax.ShapeDtypeStruct((NW * CAND,), jnp.float32),
            jax.ShapeDtypeStruct((NW * CAND,), jnp.int32),
            jax.ShapeDtypeStruct((NW * CAND,), jnp.int32),
            jax.ShapeDtypeStruct((NW * 16,), jnp.int32),
        ),
        mesh=_sc_mesh(),
        compiler_params=_SC_PARAMS,
        scratch_types=[
            pltpu.VMEM((CH,), jnp.float32),
            pltpu.VMEM((CAND,), jnp.float32),
            pltpu.VMEM((CAND,), jnp.int32),
            pltpu.VMEM((CAND,), jnp.int32),
            pltpu.VMEM((NW * NB,), jnp.int32),
            pltpu.VMEM((NB,), jnp.int32),
            pltpu.VMEM((16,), jnp.int32),
        ],
    )(flip_flat, h1, h2, h3)


def _assemble_kernel(gval_ref, gidx_ref, tidx_ref, cnt_ref, loss_ref,
                     topk_ref, dist_ref, lrec_ref):
    cnt = cnt_ref[...]
    g = cnt[:, 0:1].astype(jnp.float32)
    t = cnt[:, 1:2].astype(jnp.float32)
    g_total = cnt[0:1, 2:3].astype(jnp.float32)
    t_need = cnt[0:1, 3:4].astype(jnp.float32)
    vstar = lax.bitcast_convert_type(cnt[0:1, 4:5], jnp.float32)
    wi = lax.broadcasted_iota(jnp.int32, (NW, NW), 1).astype(jnp.float32)
    wj = lax.broadcasted_iota(jnp.int32, (NW, NW), 0).astype(jnp.float32)
    tri = (wi < wj).astype(jnp.float32)
    gpref = jnp.sum(tri * jnp.transpose(g), axis=1, keepdims=True)
    tpref = jnp.sum(tri * jnp.transpose(t), axis=1, keepdims=True)
    c2d = lax.broadcasted_iota(jnp.int32, (NW, CAND), 1).astype(jnp.float32)
    big = jnp.float32(1 << 24)
    validg = c2d < g
    rank_g = jnp.where(validg, gpref + c2d, big)
    val_g = gval_ref[...]
    idx_g = gidx_ref[...].astype(jnp.float32)
    validt = (c2d < t) & (tpref + c2d < t_need)
    rank_t = jnp.where(validt, g_total + tpref + c2d, big)
    idx_t = tidx_ref[...].astype(jnp.float32)
    p_iota = lax.broadcasted_iota(jnp.int32, (K, 1), 0).astype(jnp.float32)
    av = jnp.zeros((K, 1), jnp.float32)
    ai = jnp.zeros((K, 1), jnp.float32)
    for w in range(NW):
        ohg = (rank_g[w : w + 1, :] == p_iota).astype(jnp.float32)
        av = av + jnp.sum(ohg * val_g[w : w + 1, :], axis=1, keepdims=True)
        ai = ai + jnp.sum(ohg * idx_g[w : w + 1, :], axis=1, keepdims=True)
        oht = (rank_t[w : w + 1, :] == p_iota).astype(jnp.float32)
        av = av + jnp.sum(oht, axis=1, keepdims=True) * vstar
        ai = ai + jnp.sum(oht * idx_t[w : w + 1, :], axis=1, keepdims=True)
    vT = jnp.transpose(av)
    iT = jnp.transpose(ai)
    beats = (vT > av) | ((vT == av) & (iT < ai))
    rank = jnp.sum(beats.astype(jnp.float32), axis=1, keepdims=True)
    oh = (jnp.transpose(rank) == p_iota).astype(jnp.float32)
    topk_ref[...] = jnp.sum(oh * iT, axis=1, keepdims=True).astype(jnp.int32)
    dist_ref[0, 0] = jnp.sum(av)
    lrec_ref[0, 0] = loss_ref[0, 0] / (N * N)


def _assemble(gval, gidx, tidx, cnt, loss_sum):
    return pl.pallas_call(
        _assemble_kernel,
        in_specs=[
            pl.BlockSpec((NW, CAND), lambda: (0, 0)),
            pl.BlockSpec((NW, CAND), lambda: (0, 0)),
            pl.BlockSpec((NW, CAND), lambda: (0, 0)),
            pl.BlockSpec((NW, 16), lambda: (0, 0)),
            pl.BlockSpec(memory_space=pltpu.SMEM),
        ],
        out_specs=[
            pl.BlockSpec((K, 1), lambda: (0, 0)),
            pl.BlockSpec(memory_space=pltpu.SMEM),
            pl.BlockSpec(memory_space=pltpu.SMEM),
        ],
        out_shape=[
            jax.ShapeDtypeStruct((K, 1), jnp.int32),
            jax.ShapeDtypeStruct((1, 1), jnp.float32),
            jax.ShapeDtypeStruct((1, 1), jnp.float32),
        ],
    )(
        gval.reshape(NW, CAND),
        gidx.reshape(NW, CAND),
        tidx.reshape(NW, CAND),
        cnt.reshape(NW, 16),
        loss_sum,
    )


def _flip_points_body(idx_ref, a_hbm, idxv, valbuf, sem):
    wid = _wid()

    @pl.when(wid == 0)
    def _():
        pltpu.sync_copy(idx_ref, idxv)
        for k2 in range(4):
            pltpu.async_copy(a_hbm.at[idxv.at[k2]], valbuf, sem).wait()
            for j in range(8):
                v = valbuf[pl.ds(j * 16, 16)]
                valbuf[pl.ds(j * 16, 16)] = 1.0 - v
            pltpu.async_copy(valbuf, a_hbm.at[idxv.at[k2]], sem).wait()


def _flip_points(topk_idx, a_ref):
    pl.kernel(
        _flip_points_body,
        out_type=(),
        mesh=_sc_mesh(),
        compiler_params=_SC_PARAMS,
        scratch_types=[
            pltpu.VMEM((4, 128), jnp.int32),
            pltpu.VMEM((128,), jnp.float32),
            pltpu.SemaphoreType.DMA,
        ],
    )(topk_idx.reshape(4, 128), a_ref)


def kernel(feats, edge_index, y, q, W_enc):
    del y, q
    a_ref = jax.new_ref(jnp.zeros((N * N,), jnp.float32))
    _scatter_edges(edge_index.reshape(-1), a_ref)
    a = a_ref[...]
    flip, loss_sum = _compute_flip(feats, W_enc, a.reshape(N, N))
    flat = flip.reshape(-1)
    h1 = _hist_pass(20, flat, [])
    h2 = _hist_pass(10, flat, [h1])
    h3 = _hist_pass(0, flat, [h1, h2])
    gval, gidx, tidx, cnt = _compact(flat, h1, h2, h3)
    topk2d, dist, lrec = _assemble(gval, gidx, tidx, cnt, loss_sum)
    topk_indices = topk2d.reshape(K)
    _flip_points(topk_indices, a_ref)
    A_att = a_ref[...].reshape(N, N)
    return (A_att, flat, topk_indices, lrec.reshape(()), dist.reshape(()))


# speedup vs baseline: 12.1055x; 1.2543x over previous
import functools

import jax
import jax.numpy as jnp
from jax import lax
from jax.experimental import pallas as pl
from jax.experimental.pallas import tpu as pltpu
from jax.experimental.pallas import tpu_sc as plsc

N = 4096
D = 256
E = 65536
K = 512
RB = 128
NBLK = N // RB

NW = 32
TILE_ELEMS = N * N // NW
CH = 32768
NCHUNK = TILE_ELEMS // CH
NB = 1024
CAND = 640

_sc_mesh = functools.partial(
    plsc.VectorSubcoreMesh, core_axis_name="c", subcore_axis_name="s"
)
_SC_PARAMS = pltpu.CompilerParams(needs_layout_passes=False)


def _wid():
    return lax.axis_index("s") * 2 + lax.axis_index("c")


def _flip_kernel(feats_ref, w_ref, a_ref, flip_ref, loss_ref, h_ref):
    i = pl.program_id(0)

    @pl.when(i == 0)
    def _():
        h_ref[...] = jnp.tanh(
            lax.dot_general(feats_ref[...], w_ref[...], (((1,), (0,)), ((), ())))
        )
        loss_ref[0, 0] = 0.0

    h_blk = h_ref[pl.ds(i * RB, RB), :]
    s = lax.dot_general(h_blk, h_ref[...], (((1,), (1,)), ((), ())))
    sig = jax.nn.sigmoid(s)
    rows = i * RB + lax.broadcasted_iota(jnp.int32, (RB, N), 0)
    cols = lax.broadcasted_iota(jnp.int32, (RB, N), 1)
    sig = jnp.where(rows == cols, 0.0, sig)
    flip = jnp.abs(sig - a_ref[...])
    flip_ref[...] = flip
    loss_ref[0, 0] += jnp.sum(flip * flip)


def _compute_flip(feats, w_enc, a):
    return pl.pallas_call(
        _flip_kernel,
        grid=(NBLK,),
        in_specs=[
            pl.BlockSpec((N, D), lambda i: (0, 0)),
            pl.BlockSpec((D, D), lambda i: (0, 0)),
            pl.BlockSpec((RB, N), lambda i: (i, 0)),
        ],
        out_specs=[
            pl.BlockSpec((RB, N), lambda i: (i, 0)),
            pl.BlockSpec(memory_space=pltpu.SMEM),
        ],
        out_shape=[
            jax.ShapeDtypeStruct((N, N), jnp.float32),
            jax.ShapeDtypeStruct((1, 1), jnp.float32),
        ],
        scratch_shapes=[pltpu.VMEM((N, D), jnp.float32)],
    )(feats, w_enc, a)


def _scatter_edges_body(e_ref, a_hbm, ebuf_r, ebuf_c, idx2d, ones_v, sem):
    wid = _wid()
    base = wid * (E // NW)
    pltpu.sync_copy(e_ref.at[pl.ds(base, E // NW)], ebuf_r)
    pltpu.sync_copy(e_ref.at[pl.ds(E + base, E // NW)], ebuf_c)
    for j in range(8):
        ones_v[pl.ds(j * 16, 16)] = jnp.full((16,), 1.0, jnp.float32)
    for j in range(E // NW // 16):
        r16 = ebuf_r[pl.ds(j * 16, 16)]
        c16 = ebuf_c[pl.ds(j * 16, 16)]
        idx2d[j // 8, pl.ds((j % 8) * 16, 16)] = r16 * N + c16
    copies = []
    for k2 in range(16):
        copies.append(pltpu.async_copy(ones_v, a_hbm.at[idx2d.at[k2]], sem))
    for cp in copies:
        cp.wait()


def _scatter_edges(edge_flat, a_ref):
    pl.kernel(
        _scatter_edges_body,
        out_type=(),
        mesh=_sc_mesh(),
        compiler_params=_SC_PARAMS,
        scratch_types=[
            pltpu.VMEM((E // NW,), jnp.int32),
            pltpu.VMEM((E // NW,), jnp.int32),
            pltpu.VMEM((16, 128), jnp.int32),
            pltpu.VMEM((128,), jnp.float32),
            pltpu.SemaphoreType.DMA,
        ],
    )(edge_flat, a_ref)


def _accum_hist(hist_hbm, tmp, total):
    pltpu.sync_copy(hist_hbm, tmp)

    def body(w, _):
        def inner(j, _):
            total[pl.ds(j * 16, 16)] = total[pl.ds(j * 16, 16)] + tmp[
                pl.ds(w * NB + j * 16, 16)
            ]
            return 0

        return lax.fori_loop(0, NB // 16, inner, 0)

    lax.fori_loop(0, NW, body, 0)


def _zero(ref, n):
    def body(j, _):
        ref[pl.ds(j * 16, 16)] = jnp.zeros((16,), ref.dtype)
        return 0

    lax.fori_loop(0, n // 16, body, 0)


def _select_bucket(total, r_scalar):
    rvec = jnp.full((16,), 0, jnp.int32) + r_scalar

    def body(jj, carry):
        acc, cnt = carry
        j = NB // 16 - 1 - jj
        v = total[pl.ds(j * 16, 16)]
        rc = lax.rev(plsc.cumsum(lax.rev(v, (0,))), (0,))
        suf = rc + acc
        cnt = cnt + jnp.sum((suf >= rvec).astype(jnp.int32))
        acc = acc + jnp.sum(v)
        return acc, cnt

    _, cnt_ge = lax.fori_loop(0, NB // 16, body, (jnp.int32(0), jnp.int32(0)))
    b = cnt_ge - 1
    bvec = jnp.full((16,), 0, jnp.int32) + b

    def body2(j, s):
        v = total[pl.ds(j * 16, 16)]
        gi = j * 16 + lax.iota(jnp.int32, 16)
        s = s + jnp.sum(jnp.where(gi > bvec, v, 0))
        return s

    s_gt = lax.fori_loop(0, NB // 16, body2, jnp.int32(0))
    return b, s_gt


def _hist_pass_body(sh, nprev, flip_ref, *rest):
    prev_hbm = rest[:nprev]
    histo_hbm = rest[nprev]
    chunk, hist, tmp, total = rest[nprev + 1 : nprev + 5]
    wid = _wid()
    p = jnp.int32(0)
    r = jnp.int32(K)
    for ph in range(nprev):
        _zero(total, NB)
        _accum_hist(prev_hbm[ph], tmp, total)
        b, s_gt = _select_bucket(total, r)
        p = p * NB + b
        r = r - s_gt
    pvec = jnp.full((16,), 0, jnp.int32) + p
    _zero(hist, 16 * NB)
    laneoff = lax.iota(jnp.int32, 16) * NB
    ones16 = jnp.full((16,), 1, jnp.int32)
    base = wid * TILE_ELEMS

    def chunk_body(ci, _):
        pltpu.sync_copy(flip_ref.at[pl.ds(base + ci * CH, CH)], chunk)

        def vbody(jb, _):
            for u in range(8):
                x = chunk[pl.ds(jb * 128 + u * 16, 16)]
                bits = plsc.bitcast(x, jnp.int32)
                pred = lax.shift_right_logical(bits, sh + 10) == pvec
                bucket = lax.bitwise_and(
                    lax.shift_right_logical(bits, sh), NB - 1)
                plsc.addupdate_scatter(
                    hist, [bucket + laneoff], ones16, mask=pred)
            return 0

        lax.fori_loop(0, CH // 128, vbody, 0)
        return 0

    lax.fori_loop(0, NCHUNK, chunk_body, 0)

    def fold(j, _):
        s = hist[pl.ds(j * 16, 16)]
        for l in range(1, 16):
            s = s + hist[pl.ds(l * NB + j * 16, 16)]
        tmp[pl.ds(j * 16, 16)] = s
        return 0

    lax.fori_loop(0, NB // 16, fold, 0)
    pltpu.sync_copy(tmp.at[pl.ds(0, NB)], histo_hbm.at[pl.ds(wid * NB, NB)])


def _hist_pass(sh, flip_flat, prev_hists):
    nprev = len(prev_hists)
    body = functools.partial(_hist_pass_body, sh, nprev)
    return pl.kernel(
        body,
        out_type=jax.ShapeDtypeStruct((NW * NB,), jnp.int32),
        mesh=_sc_mesh(),
        compiler_params=_SC_PARAMS,
        scratch_types=[
            pltpu.VMEM((CH,), jnp.float32),
            pltpu.VMEM((16 * NB,), jnp.int32),
            pltpu.VMEM((NW * NB,), jnp.int32),
            pltpu.VMEM((NB,), jnp.int32),
        ],
    )(flip_flat, *prev_hists)


def _compact_body(flip_ref, h1, h2, h3, gval_o, gidx_o, tidx_o, cnt_o,
                  chunk, gval, gidx, tidx, tmp, total, cbuf):
    wid = _wid()
    p = jnp.int32(0)
    r = jnp.int32(K)
    s_total = jnp.int32(0)
    for hh in (h1, h2, h3):
        _zero(total, NB)
        _accum_hist(hh, tmp, total)
        b, s_gt = _select_bucket(total, r)
        p = p * NB + b
        r = r - s_gt
        s_total = s_total + s_gt
    vstar_bits = p
    g_total = s_total
    t_need = jnp.int32(K) - g_total
    vstar = plsc.bitcast(jnp.full((16,), 0, jnp.int32) + vstar_bits, jnp.float32)
    base = wid * TILE_ELEMS
    iota16 = lax.iota(jnp.int32, 16)

    def chunk_body(ci, carry):
        pltpu.sync_copy(flip_ref.at[pl.ds(base + ci * CH, CH)], chunk)

        def vbody(jb, carry):
            goffv, toffv = carry
            for u in range(4):
                x = chunk[pl.ds(jb * 64 + u * 16, 16)]
                m_gt = x > vstar
                m_eq = x == vstar
                c_gt = plsc.all_reduce_population_count(m_gt)
                c_eq = plsc.all_reduce_population_count(m_eq)
                gidxv = base + ci * CH + jb * 64 + u * 16 + iota16
                dg = goffv + plsc.cumsum(m_gt.astype(jnp.int32)) - 1
                mg = m_gt & (dg < CAND - 16)
                plsc.store_scatter(gval, [dg], x, mask=mg)
                plsc.store_scatter(gidx, [dg], gidxv, mask=mg)
                dt = toffv + plsc.cumsum(m_eq.astype(jnp.int32)) - 1
                mt = m_eq & (dt < K)
                plsc.store_scatter(tidx, [dt], gidxv, mask=mt)
                goffv = goffv + c_gt
                toffv = toffv + c_eq
            return goffv, toffv

        return lax.fori_loop(0, CH // 64, vbody, carry)

    z16 = jnp.zeros((16,), jnp.int32)
    goff, toff = lax.fori_loop(0, NCHUNK, chunk_body, (z16, z16))
    cvals = jnp.where(
        iota16 == 0, goff,
        jnp.where(iota16 == 1, jnp.minimum(toff, K),
                  jnp.where(iota16 == 2, g_total,
                            jnp.where(iota16 == 3, t_need, vstar_bits))))
    cbuf[pl.ds(0, 16)] = cvals
    pltpu.sync_copy(gval, gval_o.at[pl.ds(wid * CAND, CAND)])
    pltpu.sync_copy(gidx, gidx_o.at[pl.ds(wid * CAND, CAND)])
    pltpu.sync_copy(tidx, tidx_o.at[pl.ds(wid * CAND, CAND)])
    pltpu.sync_copy(cbuf, cnt_o.at[pl.ds(wid * 16, 16)])


def _compact(flip_flat, h1, h2, h3):
    return pl.kernel(
        _compact_body,
        out_type=(
            jax.ShapeDtypeStruct((NW * CAND,), jnp.float32),
            jax.ShapeDtypeStruct((NW * CAND,), jnp.int32),
            jax.ShapeDtypeStruct((NW * CAND,), jnp.int32),
            jax.ShapeDtypeStruct((NW * 16,), jnp.int32),
        ),
        mesh=_sc_mesh(),
        compiler_params=_SC_PARAMS,
        scratch_types=[
            pltpu.VMEM((CH,), jnp.float32),
            pltpu.VMEM((CAND,), jnp.float32),
            pltpu.VMEM((CAND,), jnp.int32),
            pltpu.VMEM((CAND,), jnp.int32),
            pltpu.VMEM((NW * NB,), jnp.int32),
            pltpu.VMEM((NB,), jnp.int32),
            pltpu.VMEM((16,), jnp.int32),
        ],
    )(flip_flat, h1, h2, h3)


def _assemble_kernel(gval_ref, gidx_ref, tidx_ref, cnt_ref, loss_ref,
                     topk_ref, dist_ref, lrec_ref):
    cnt = cnt_ref[...]
    g = cnt[:, 0:1].astype(jnp.float32)
    t = cnt[:, 1:2].astype(jnp.float32)
    g_total = cnt[0:1, 2:3].astype(jnp.float32)
    t_need = cnt[0:1, 3:4].astype(jnp.float32)
    vstar = lax.bitcast_convert_type(cnt[0:1, 4:5], jnp.float32)
    wi = lax.broadcasted_iota(jnp.int32, (NW, NW), 1).astype(jnp.float32)
    wj = lax.broadcasted_iota(jnp.int32, (NW, NW), 0).astype(jnp.float32)
    tri = (wi < wj).astype(jnp.float32)
    gpref = jnp.sum(tri * jnp.transpose(g), axis=1, keepdims=True)
    tpref = jnp.sum(tri * jnp.transpose(t), axis=1, keepdims=True)
    c2d = lax.broadcasted_iota(jnp.int32, (NW, CAND), 1).astype(jnp.float32)
    big = jnp.float32(1 << 24)
    validg = c2d < g
    rank_g = jnp.where(validg, gpref + c2d, big)
    val_g = gval_ref[...]
    idx_g = gidx_ref[...].astype(jnp.float32)
    validt = (c2d < t) & (tpref + c2d < t_need)
    rank_t = jnp.where(validt, g_total + tpref + c2d, big)
    idx_t = tidx_ref[...].astype(jnp.float32)
    p_iota = lax.broadcasted_iota(jnp.int32, (K, 1), 0).astype(jnp.float32)
    av = jnp.zeros((K, 1), jnp.float32)
    ai = jnp.zeros((K, 1), jnp.float32)
    for w in range(NW):
        ohg = (rank_g[w : w + 1, :] == p_iota).astype(jnp.float32)
        av = av + jnp.sum(ohg * val_g[w : w + 1, :], axis=1, keepdims=True)
        ai = ai + jnp.sum(ohg * idx_g[w : w + 1, :], axis=1, keepdims=True)
        oht = (rank_t[w : w + 1, :] == p_iota).astype(jnp.float32)
        av = av + jnp.sum(oht, axis=1, keepdims=True) * vstar
        ai = ai + jnp.sum(oht * idx_t[w : w + 1, :], axis=1, keepdims=True)
    vT = jnp.transpose(av)
    iT = jnp.transpose(ai)
    beats = (vT > av) | ((vT == av) & (iT < ai))
    rank = jnp.sum(beats.astype(jnp.float32), axis=1, keepdims=True)
    oh = (jnp.transpose(rank) == p_iota).astype(jnp.float32)
    topk_ref[...] = jnp.sum(oh * iT, axis=1, keepdims=True).astype(jnp.int32)
    dist_ref[0, 0] = jnp.sum(av)
    lrec_ref[0, 0] = loss_ref[0, 0] / (N * N)


def _assemble(gval, gidx, tidx, cnt, loss_sum):
    return pl.pallas_call(
        _assemble_kernel,
        in_specs=[
            pl.BlockSpec((NW, CAND), lambda: (0, 0)),
            pl.BlockSpec((NW, CAND), lambda: (0, 0)),
            pl.BlockSpec((NW, CAND), lambda: (0, 0)),
            pl.BlockSpec((NW, 16), lambda: (0, 0)),
            pl.BlockSpec(memory_space=pltpu.SMEM),
        ],
        out_specs=[
            pl.BlockSpec((K, 1), lambda: (0, 0)),
            pl.BlockSpec(memory_space=pltpu.SMEM),
            pl.BlockSpec(memory_space=pltpu.SMEM),
        ],
        out_shape=[
            jax.ShapeDtypeStruct((K, 1), jnp.int32),
            jax.ShapeDtypeStruct((1, 1), jnp.float32),
            jax.ShapeDtypeStruct((1, 1), jnp.float32),
        ],
    )(
        gval.reshape(NW, CAND),
        gidx.reshape(NW, CAND),
        tidx.reshape(NW, CAND),
        cnt.reshape(NW, 16),
        loss_sum,
    )


def _flip_points_body(idx_ref, a_hbm, idxv, valbuf, sem):
    wid = _wid()

    @pl.when(wid == 0)
    def _():
        pltpu.sync_copy(idx_ref, idxv)
        for k2 in range(4):
            pltpu.async_copy(a_hbm.at[idxv.at[k2]], valbuf, sem).wait()
            for j in range(8):
                v = valbuf[pl.ds(j * 16, 16)]
                valbuf[pl.ds(j * 16, 16)] = 1.0 - v
            pltpu.async_copy(valbuf, a_hbm.at[idxv.at[k2]], sem).wait()


def _flip_points(topk_idx, a_ref):
    pl.kernel(
        _flip_points_body,
        out_type=(),
        mesh=_sc_mesh(),
        compiler_params=_SC_PARAMS,
        scratch_types=[
            pltpu.VMEM((4, 128), jnp.int32),
            pltpu.VMEM((128,), jnp.float32),
            pltpu.SemaphoreType.DMA,
        ],
    )(topk_idx.reshape(4, 128), a_ref)


def kernel(feats, edge_index, y, q, W_enc):
    del y, q
    a_ref = jax.new_ref(jnp.zeros((N * N,), jnp.float32))
    _scatter_edges(edge_index.reshape(-1), a_ref)
    a = a_ref[...]
    flip, loss_sum = _compute_flip(feats, W_enc, a.reshape(N, N))
    flat = flip.reshape(-1)
    h1 = _hist_pass(20, flat, [])
    h2 = _hist_pass(10, flat, [h1])
    h3 = _hist_pass(0, flat, [h1, h2])
    gval, gidx, tidx, cnt = _compact(flat, h1, h2, h3)
    topk2d, dist, lrec = _assemble(gval, gidx, tidx, cnt, loss_sum)
    topk_indices = topk2d.reshape(K)
    _flip_points(topk_indices, a_ref)
    A_att = a_ref[...].reshape(N, N)
    return (A_att, flat, topk_indices, lrec.reshape(()), dist.reshape(()))


# speedup vs baseline: 12.4160x; 1.0256x over previous
import functools

import jax
import jax.numpy as jnp
from jax import lax
from jax.experimental import pallas as pl
from jax.experimental.pallas import tpu as pltpu
from jax.experimental.pallas import tpu_sc as plsc

N = 4096
D = 256
E = 65536
K = 512
RB = 128
NBLK = N // RB

NW = 32
TILE_ELEMS = N * N // NW
CH = 32768
NCHUNK = TILE_ELEMS // CH
NB = 1024
CAND = 640

_sc_mesh = functools.partial(
    plsc.VectorSubcoreMesh, core_axis_name="c", subcore_axis_name="s"
)
_SC_PARAMS = pltpu.CompilerParams(needs_layout_passes=False)


def _wid():
    return lax.axis_index("s") * 2 + lax.axis_index("c")


def _flip_kernel(feats_ref, w_ref, a_ref, flip_ref, loss_ref, h_ref):
    i = pl.program_id(0)

    @pl.when(i == 0)
    def _():
        h_ref[...] = jnp.tanh(
            lax.dot_general(feats_ref[...], w_ref[...], (((1,), (0,)), ((), ())))
        )
        loss_ref[0, 0] = 0.0

    h_blk = h_ref[pl.ds(i * RB, RB), :]
    s = lax.dot_general(h_blk, h_ref[...], (((1,), (1,)), ((), ())))
    sig = jax.nn.sigmoid(s)
    rows = i * RB + lax.broadcasted_iota(jnp.int32, (RB, N), 0)
    cols = lax.broadcasted_iota(jnp.int32, (RB, N), 1)
    sig = jnp.where(rows == cols, 0.0, sig)
    flip = jnp.abs(sig - a_ref[...])
    flip_ref[...] = flip
    loss_ref[0, 0] += jnp.sum(flip * flip)


def _compute_flip(feats, w_enc, a):
    return pl.pallas_call(
        _flip_kernel,
        grid=(NBLK,),
        in_specs=[
            pl.BlockSpec((N, D), lambda i: (0, 0)),
            pl.BlockSpec((D, D), lambda i: (0, 0)),
            pl.BlockSpec((RB, N), lambda i: (i, 0)),
        ],
        out_specs=[
            pl.BlockSpec((RB, N), lambda i: (i, 0)),
            pl.BlockSpec(memory_space=pltpu.SMEM),
        ],
        out_shape=[
            jax.ShapeDtypeStruct((N, N), jnp.float32),
            jax.ShapeDtypeStruct((1, 1), jnp.float32),
        ],
        scratch_shapes=[pltpu.VMEM((N, D), jnp.float32)],
    )(feats, w_enc, a)


def _scatter_edges_body(e_ref, a_hbm, ebuf_r, ebuf_c, idx2d, ones_v, sem):
    wid = _wid()
    base = wid * (E // NW)
    pltpu.sync_copy(e_ref.at[pl.ds(base, E // NW)], ebuf_r)
    pltpu.sync_copy(e_ref.at[pl.ds(E + base, E // NW)], ebuf_c)
    for j in range(8):
        ones_v[pl.ds(j * 16, 16)] = jnp.full((16,), 1.0, jnp.float32)
    for j in range(E // NW // 16):
        r16 = ebuf_r[pl.ds(j * 16, 16)]
        c16 = ebuf_c[pl.ds(j * 16, 16)]
        idx2d[j // 8, pl.ds((j % 8) * 16, 16)] = r16 * N + c16
    copies = []
    for k2 in range(16):
        copies.append(pltpu.async_copy(ones_v, a_hbm.at[idx2d.at[k2]], sem))
    for cp in copies:
        cp.wait()


def _scatter_edges(edge_flat, a_ref):
    pl.kernel(
        _scatter_edges_body,
        out_type=(),
        mesh=_sc_mesh(),
        compiler_params=_SC_PARAMS,
        scratch_types=[
            pltpu.VMEM((E // NW,), jnp.int32),
            pltpu.VMEM((E // NW,), jnp.int32),
            pltpu.VMEM((16, 128), jnp.int32),
            pltpu.VMEM((128,), jnp.float32),
            pltpu.SemaphoreType.DMA,
        ],
    )(edge_flat, a_ref)


def _accum_hist(hist_hbm, tmp, total):
    pltpu.sync_copy(hist_hbm, tmp)

    def body(w, _):
        def inner(j, _):
            total[pl.ds(j * 16, 16)] = total[pl.ds(j * 16, 16)] + tmp[
                pl.ds(w * NB + j * 16, 16)
            ]
            return 0

        return lax.fori_loop(0, NB // 16, inner, 0)

    lax.fori_loop(0, NW, body, 0)


def _zero(ref, n):
    def body(j, _):
        ref[pl.ds(j * 16, 16)] = jnp.zeros((16,), ref.dtype)
        return 0

    lax.fori_loop(0, n // 16, body, 0)


def _select_bucket(total, r_scalar):
    rvec = jnp.full((16,), 0, jnp.int32) + r_scalar

    def body(jj, carry):
        acc, cnt = carry
        j = NB // 16 - 1 - jj
        v = total[pl.ds(j * 16, 16)]
        rc = lax.rev(plsc.cumsum(lax.rev(v, (0,))), (0,))
        suf = rc + acc
        cnt = cnt + jnp.sum((suf >= rvec).astype(jnp.int32))
        acc = acc + jnp.sum(v)
        return acc, cnt

    _, cnt_ge = lax.fori_loop(0, NB // 16, body, (jnp.int32(0), jnp.int32(0)))
    b = cnt_ge - 1
    bvec = jnp.full((16,), 0, jnp.int32) + b

    def body2(j, s):
        v = total[pl.ds(j * 16, 16)]
        gi = j * 16 + lax.iota(jnp.int32, 16)
        s = s + jnp.sum(jnp.where(gi > bvec, v, 0))
        return s

    s_gt = lax.fori_loop(0, NB // 16, body2, jnp.int32(0))
    return b, s_gt


def _hist_pass_body(sh, nprev, flip_ref, *rest):
    prev_hbm = rest[:nprev]
    histo_hbm = rest[nprev]
    chunk, hist, tmp, total = rest[nprev + 1 : nprev + 5]
    wid = _wid()
    p = jnp.int32(0)
    r = jnp.int32(K)
    for ph in range(nprev):
        _zero(total, NB)
        _accum_hist(prev_hbm[ph], tmp, total)
        b, s_gt = _select_bucket(total, r)
        p = p * NB + b
        r = r - s_gt
    pvec = jnp.full((16,), 0, jnp.int32) + p
    _zero(hist, 16 * NB)
    iota16 = lax.iota(jnp.int32, 16)
    ones16 = jnp.full((16,), 1, jnp.int32)
    base = wid * TILE_ELEMS

    def chunk_body(ci, _):
        pltpu.sync_copy(flip_ref.at[pl.ds(base + ci * CH, CH)], chunk)

        def vbody(jb, _):
            for u in range(8):
                x = chunk[pl.ds(jb * 128 + u * 16, 16)]
                bits = plsc.bitcast(x, jnp.int32)
                pred = lax.shift_right_logical(bits, sh + 10) == pvec
                bucket = lax.bitwise_and(
                    lax.shift_right_logical(bits, sh), NB - 1)
                plsc.addupdate_scatter(
                    hist, [bucket * 16 + iota16], ones16, mask=pred)
            return 0

        lax.fori_loop(0, CH // 128, vbody, 0)
        return 0

    lax.fori_loop(0, NCHUNK, chunk_body, 0)

    def fold(j, _):
        bidx = (j * 16 + iota16) * 16
        s = plsc.load_gather(hist, [bidx])
        for l in range(1, 16):
            s = s + plsc.load_gather(hist, [bidx + l])
        tmp[pl.ds(j * 16, 16)] = s
        return 0

    lax.fori_loop(0, NB // 16, fold, 0)
    pltpu.sync_copy(tmp.at[pl.ds(0, NB)], histo_hbm.at[pl.ds(wid * NB, NB)])


def _hist_pass(sh, flip_flat, prev_hists):
    nprev = len(prev_hists)
    body = functools.partial(_hist_pass_body, sh, nprev)
    return pl.kernel(
        body,
        out_type=jax.ShapeDtypeStruct((NW * NB,), jnp.int32),
        mesh=_sc_mesh(),
        compiler_params=_SC_PARAMS,
        scratch_types=[
            pltpu.VMEM((CH,), jnp.float32),
            pltpu.VMEM((16 * NB,), jnp.int32),
            pltpu.VMEM((NW * NB,), jnp.int32),
            pltpu.VMEM((NB,), jnp.int32),
        ],
    )(flip_flat, *prev_hists)


def _compact_body(flip_ref, h1, h2, h3, gval_o, gidx_o, tidx_o, cnt_o,
                  chunk, gval, gidx, tidx, tmp, total, cbuf):
    wid = _wid()
    p = jnp.int32(0)
    r = jnp.int32(K)
    s_total = jnp.int32(0)
    for hh in (h1, h2, h3):
        _zero(total, NB)
        _accum_hist(hh, tmp, total)
        b, s_gt = _select_bucket(total, r)
        p = p * NB + b
        r = r - s_gt
        s_total = s_total + s_gt
    vstar_bits = p
    g_total = s_total
    t_need = jnp.int32(K) - g_total
    vstar = plsc.bitcast(jnp.full((16,), 0, jnp.int32) + vstar_bits, jnp.float32)
    base = wid * TILE_ELEMS
    iota16 = lax.iota(jnp.int32, 16)

    def chunk_body(ci, carry):
        pltpu.sync_copy(flip_ref.at[pl.ds(base + ci * CH, CH)], chunk)

        def vbody(jb, carry):
            goffv, toffv = carry
            for u in range(4):
                x = chunk[pl.ds(jb * 64 + u * 16, 16)]
                m_gt = x > vstar
                m_eq = x == vstar
                c_gt = plsc.all_reduce_population_count(m_gt)
                c_eq = plsc.all_reduce_population_count(m_eq)
                gidxv = base + ci * CH + jb * 64 + u * 16 + iota16
                dg = goffv + plsc.cumsum(m_gt.astype(jnp.int32)) - 1
                mg = m_gt & (dg < CAND - 16)
                plsc.store_scatter(gval, [dg], x, mask=mg)
                plsc.store_scatter(gidx, [dg], gidxv, mask=mg)
                dt = toffv + plsc.cumsum(m_eq.astype(jnp.int32)) - 1
                mt = m_eq & (dt < K)
                plsc.store_scatter(tidx, [dt], gidxv, mask=mt)
                goffv = goffv + c_gt
                toffv = toffv + c_eq
            return goffv, toffv

        return lax.fori_loop(0, CH // 64, vbody, carry)

    z16 = jnp.zeros((16,), jnp.int32)
    goff, toff = lax.fori_loop(0, NCHUNK, chunk_body, (z16, z16))
    cvals = jnp.where(
        iota16 == 0, goff,
        jnp.where(iota16 == 1, jnp.minimum(toff, K),
                  jnp.where(iota16 == 2, g_total,
                            jnp.where(iota16 == 3, t_need, vstar_bits))))
    cbuf[pl.ds(0, 16)] = cvals
    pltpu.sync_copy(gval, gval_o.at[pl.ds(wid * CAND, CAND)])
    pltpu.sync_copy(gidx, gidx_o.at[pl.ds(wid * CAND, CAND)])
    pltpu.sync_copy(tidx, tidx_o.at[pl.ds(wid * CAND, CAND)])
    pltpu.sync_copy(cbuf, cnt_o.at[pl.ds(wid * 16, 16)])


def _compact(flip_flat, h1, h2, h3):
    return pl.kernel(
        _compact_body,
        out_type=(
            jax.ShapeDtypeStruct((NW * CAND,), jnp.float32),
            jax.ShapeDtypeStruct((NW * CAND,), jnp.int32),
            jax.ShapeDtypeStruct((NW * CAND,), jnp.int32),
            jax.ShapeDtypeStruct((NW * 16,), jnp.int32),
        ),
        mesh=_sc_mesh(),
        compiler_params=_SC_PARAMS,
        scratch_types=[
            pltpu.VMEM((CH,), jnp.float32),
            pltpu.VMEM((CAND,), jnp.float32),
            pltpu.VMEM((CAND,), jnp.int32),
            pltpu.VMEM((CAND,), jnp.int32),
            pltpu.VMEM((NW * NB,), jnp.int32),
            pltpu.VMEM((NB,), jnp.int32),
            pltpu.VMEM((16,), jnp.int32),
        ],
    )(flip_flat, h1, h2, h3)


def _assemble_kernel(gval_ref, gidx_ref, tidx_ref, cnt_ref, loss_ref,
                     topk_ref, dist_ref, lrec_ref):
    cnt = cnt_ref[...]
    g = cnt[:, 0:1].astype(jnp.float32)
    t = cnt[:, 1:2].astype(jnp.float32)
    g_total = cnt[0:1, 2:3].astype(jnp.float32)
    t_need = cnt[0:1, 3:4].astype(jnp.float32)
    vstar = lax.bitcast_convert_type(cnt[0:1, 4:5], jnp.float32)
    wi = lax.broadcasted_iota(jnp.int32, (NW, NW), 1).astype(jnp.float32)
    wj = lax.broadcasted_iota(jnp.int32, (NW, NW), 0).astype(jnp.float32)
    tri = (wi < wj).astype(jnp.float32)
    gpref = jnp.sum(tri * jnp.transpose(g), axis=1, keepdims=True)
    tpref = jnp.sum(tri * jnp.transpose(t), axis=1, keepdims=True)
    c2d = lax.broadcasted_iota(jnp.int32, (NW, CAND), 1).astype(jnp.float32)
    big = jnp.float32(1 << 24)
    validg = c2d < g
    rank_g = jnp.where(validg, gpref + c2d, big)
    val_g = gval_ref[...]
    idx_g = gidx_ref[...].astype(jnp.float32)
    validt = (c2d < t) & (tpref + c2d < t_need)
    rank_t = jnp.where(validt, g_total + tpref + c2d, big)
    idx_t = tidx_ref[...].astype(jnp.float32)
    p_iota = lax.broadcasted_iota(jnp.int32, (K, 1), 0).astype(jnp.float32)
    av = jnp.zeros((K, 1), jnp.float32)
    ai = jnp.zeros((K, 1), jnp.float32)
    for w in range(NW):
        ohg = (rank_g[w : w + 1, :] == p_iota).astype(jnp.float32)
        av = av + jnp.sum(ohg * val_g[w : w + 1, :], axis=1, keepdims=True)
        ai = ai + jnp.sum(ohg * idx_g[w : w + 1, :], axis=1, keepdims=True)
        oht = (rank_t[w : w + 1, :] == p_iota).astype(jnp.float32)
        av = av + jnp.sum(oht, axis=1, keepdims=True) * vstar
        ai = ai + jnp.sum(oht * idx_t[w : w + 1, :], axis=1, keepdims=True)
    vT = jnp.transpose(av)
    iT = jnp.transpose(ai)
    beats = (vT > av) | ((vT == av) & (iT < ai))
    rank = jnp.sum(beats.astype(jnp.float32), axis=1, keepdims=True)
    oh = (jnp.transpose(rank) == p_iota).astype(jnp.float32)
    topk_ref[...] = jnp.sum(oh * iT, axis=1, keepdims=True).astype(jnp.int32)
    dist_ref[0, 0] = jnp.sum(av)
    lrec_ref[0, 0] = loss_ref[0, 0] / (N * N)


def _assemble(gval, gidx, tidx, cnt, loss_sum):
    return pl.pallas_call(
        _assemble_kernel,
        in_specs=[
            pl.BlockSpec((NW, CAND), lambda: (0, 0)),
            pl.BlockSpec((NW, CAND), lambda: (0, 0)),
            pl.BlockSpec((NW, CAND), lambda: (0, 0)),
            pl.BlockSpec((NW, 16), lambda: (0, 0)),
            pl.BlockSpec(memory_space=pltpu.SMEM),
        ],
        out_specs=[
            pl.BlockSpec((K, 1), lambda: (0, 0)),
            pl.BlockSpec(memory_space=pltpu.SMEM),
            pl.BlockSpec(memory_space=pltpu.SMEM),
        ],
        out_shape=[
            jax.ShapeDtypeStruct((K, 1), jnp.int32),
            jax.ShapeDtypeStruct((1, 1), jnp.float32),
            jax.ShapeDtypeStruct((1, 1), jnp.float32),
        ],
    )(
        gval.reshape(NW, CAND),
        gidx.reshape(NW, CAND),
        tidx.reshape(NW, CAND),
        cnt.reshape(NW, 16),
        loss_sum,
    )


def _flip_points_body(idx_ref, a_hbm, idxv, valbuf, sem):
    wid = _wid()

    @pl.when(wid == 0)
    def _():
        pltpu.sync_copy(idx_ref, idxv)
        for k2 in range(4):
            pltpu.async_copy(a_hbm.at[idxv.at[k2]], valbuf, sem).wait()
            for j in range(8):
                v = valbuf[pl.ds(j * 16, 16)]
                valbuf[pl.ds(j * 16, 16)] = 1.0 - v
            pltpu.async_copy(valbuf, a_hbm.at[idxv.at[k2]], sem).wait()


def _flip_points(topk_idx, a_ref):
    pl.kernel(
        _flip_points_body,
        out_type=(),
        mesh=_sc_mesh(),
        compiler_params=_SC_PARAMS,
        scratch_types=[
            pltpu.VMEM((4, 128), jnp.int32),
            pltpu.VMEM((128,), jnp.float32),
            pltpu.SemaphoreType.DMA,
        ],
    )(topk_idx.reshape(4, 128), a_ref)


def kernel(feats, edge_index, y, q, W_enc):
    del y, q
    a_ref = jax.new_ref(jnp.zeros((N * N,), jnp.float32))
    _scatter_edges(edge_index.reshape(-1), a_ref)
    a = a_ref[...]
    flip, loss_sum = _compute_flip(feats, W_enc, a.reshape(N, N))
    flat = flip.reshape(-1)
    h1 = _hist_pass(20, flat, [])
    h2 = _hist_pass(10, flat, [h1])
    h3 = _hist_pass(0, flat, [h1, h2])
    gval, gidx, tidx, cnt = _compact(flat, h1, h2, h3)
    topk2d, dist, lrec = _assemble(gval, gidx, tidx, cnt, loss_sum)
    topk_indices = topk2d.reshape(K)
    _flip_points(topk_indices, a_ref)
    A_att = a_ref[...].reshape(N, N)
    return (A_att, flat, topk_indices, lrec.reshape(()), dist.reshape(()))
